# baseline (speedup 1.0000x reference)
import functools

import jax
import jax.numpy as jnp
from jax import lax
from jax.experimental import pallas as pl
from jax.experimental.pallas import tpu as pltpu
from jax.experimental.pallas import tpu_sc as plsc

N = 50000
E = 800000
B = 128
NSUB = 16
NCORE = 2
N_ACC = 50048
NZ = N_ACC // NSUB
NCH = E // B
CH_HALF = -(-(NCH // NCORE) // NSUB)
CH_FULL = -(-NCH // NSUB)
G = 48
NCH_PAD = 6400

RB = 5000
GRID = N // RB


@functools.lru_cache(maxsize=None)
def _make_agg(w, full, hist=False):
    mesh = plsc.VectorSubcoreMesh(core_axis_name="c", subcore_axis_name="s")
    per = NCH // NCORE if not full else NCH
    lo = per // NSUB
    extra = per - lo * NSUB
    g = 128 if w <= 8 else G
    ngrp = -(-(lo + 1) // g)

    out_type = jax.ShapeDtypeStruct((NCORE, N_ACC, w), jnp.float32)
    if hist:
        out_type = [out_type,
                    jax.ShapeDtypeStruct((2, NCH_PAD, B), jnp.int32),
                    jax.ShapeDtypeStruct((1, NCH_PAD, B), jnp.int32)]

    @functools.partial(
        pl.kernel,
        out_type=out_type,
        mesh=mesh,
        scratch_types=[
            pltpu.VMEM((g, B), jnp.int32),
            pltpu.VMEM((g, B), jnp.int32),
            [pltpu.VMEM((B, w), jnp.float32)] * 4,
            pltpu.VMEM_SHARED((N_ACC, w), jnp.float32),
            [pltpu.SemaphoreType.DMA] * 4,
            [pltpu.SemaphoreType.DMA] * 4,
        ],
        compiler_params=pltpu.CompilerParams(use_tc_tiling_on_sc=False),
    )
    def agg(*refs):
        if hist:
            table_hbm, sidx_hbm, didx_hbm, zeros_hbm, \
                out_hbm, sidx_out, didx_out, \
                sidx, didx, rows, acc, gsem, ssem = refs
        else:
            table_hbm, sidx_hbm, didx_hbm, zeros_hbm, out_hbm, \
                sidx, didx, rows, acc, gsem, ssem = refs
        c = lax.axis_index("c")
        s = lax.axis_index("s")
        nch = lo + jnp.where(s < extra, 1, 0)
        base = s * lo + jnp.minimum(s, extra) + (0 if full else c * per)
        csel = c if full else 0

        pltpu.sync_copy(zeros_hbm, acc.at[pl.ds(s * NZ, NZ)])
        if hist:
            pltpu.sync_copy(table_hbm, rows[0])
            wid = c * NSUB + s
            cp = NCH_PAD // (NCORE * NSUB) // 2
            for h in range(2):
                off = wid * 2 * cp + h * cp
                pltpu.sync_copy(didx_hbm.at[0, pl.ds(off, cp)],
                                didx.at[pl.ds(0, cp)])
                pltpu.sync_copy(didx.at[pl.ds(0, cp)],
                                didx_out.at[0, pl.ds(off, cp)])
                for p01 in range(2):
                    pltpu.sync_copy(sidx_hbm.at[p01, pl.ds(off, cp)],
                                    sidx.at[pl.ds(0, cp)])
                    pltpu.sync_copy(sidx.at[pl.ds(0, cp)],
                                    sidx_out.at[p01, pl.ds(off, cp)])
        plsc.subcore_barrier()

        def step(k, cnt, p):
            q = (p + 3) % 4
            pltpu.make_async_copy(
                table_hbm.at[pl.ds(0, B)], rows[p], gsem[p]).wait()
            pltpu.async_copy(rows[p], acc.at[didx.at[k]], ssem[p], add=True)

            @pl.when(k >= 1)
            def _():
                pltpu.make_async_copy(
                    rows[q], acc.at[didx.at[k - 1]], ssem[q]).wait()

            @pl.when(k + 3 < cnt)
            def _():
                pltpu.async_copy(
                    table_hbm.at[sidx.at[k + 3]], rows[q], gsem[q])

        def hist_step(k, p):
            pltpu.async_copy(rows[0], acc.at[didx.at[k]], ssem[p], add=True)

            @pl.when(k >= 3)
            def _():
                pltpu.make_async_copy(
                    rows[0], acc.at[didx.at[k - 3]], ssem[(p + 1) % 4]).wait()

        @pl.loop(0, ngrp)
        def _(gi):
            cnt = jnp.minimum(g, nch - gi * g)

            @pl.when(cnt > 0)
            def _():
                pltpu.sync_copy(didx_hbm.at[0, pl.ds(base + gi * g, g)], didx)
                if hist:
                    @pl.loop(0, cnt)
                    def _(k):
                        for p in range(4):
                            @pl.when(lax.rem(k, 4) == p)
                            def _():
                                hist_step(k, p)

                    @pl.loop(jnp.maximum(cnt - 3, 0), cnt)
                    def _(r):
                        for p in range(4):
                            @pl.when(lax.rem(r, 4) == p)
                            def _():
                                pltpu.make_async_copy(
                                    rows[0], acc.at[didx.at[r]],
                                    ssem[p]).wait()
                else:
                    pltpu.sync_copy(
                        sidx_hbm.at[csel, pl.ds(base + gi * g, g)], sidx)
                    for r in range(3):
                        @pl.when(r < cnt)
                        def _():
                            pltpu.async_copy(
                                table_hbm.at[sidx.at[r]], rows[r], gsem[r])

                    @pl.loop(0, cnt)
                    def _(k):
                        for p in range(4):
                            @pl.when(lax.rem(k, 4) == p)
                            def _():
                                step(k, cnt, p)

                    for p in range(4):
                        @pl.when(lax.rem(cnt - 1, 4) == p)
                        def _():
                            pltpu.make_async_copy(
                                rows[p], acc.at[didx.at[cnt - 1]],
                                ssem[p]).wait()

        plsc.subcore_barrier()
        pltpu.sync_copy(acc.at[pl.ds(s * NZ, NZ)],
                        out_hbm.at[c, pl.ds(s * NZ, NZ)])

    return agg


def _agg(table, sidx, didx, w, full=False, hist=False):
    zeros = jnp.zeros((NZ, w), jnp.float32)
    return _make_agg(w, full, hist)(table, sidx, didx, zeros)


def _row_spec(w):
    return pl.BlockSpec((RB, w), lambda i: (i, 0))


def _fix_spec(shape):
    return pl.BlockSpec(shape, lambda i: (0,) * len(shape))


def _pair_spec(w):
    return pl.BlockSpec((2, RB, w), lambda i: (0, i, 0))


def _s0_body(dp_ref, x_ref, t1_ref, dinv8_ref):
    deg = dp_ref[0, :, 0:1] + dp_ref[1, :, 0:1] + 1.0
    di = lax.rsqrt(deg)
    dinv8_ref[...] = jnp.broadcast_to(di, (RB, 8))
    t = di * x_ref[...]
    t1_ref[...] = jnp.concatenate([t, jnp.zeros((RB, 2), jnp.float32)], axis=1)


def _stage0(dp, x):
    return pl.pallas_call(
        _s0_body,
        grid=(GRID,),
        in_specs=[_pair_spec(8), _row_spec(6)],
        out_specs=[_row_spec(8), _row_spec(8)],
        out_shape=[jax.ShapeDtypeStruct((N, 8), jnp.float32),
                   jax.ShapeDtypeStruct((N, 8), jnp.float32)],
    )(dp, x)


def _layer_body(a_ref, tp_ref, dinv8_ref, w_ref, b_ref, out_ref):
    di = dinv8_ref[:, 0:1]
    u = (a_ref[0] + a_ref[1] + tp_ref[...]) * di
    h = jnp.dot(u, w_ref[...], preferred_element_type=jnp.float32) + b_ref[...]
    t = jnp.maximum(h, 0.0) * di
    if out_ref.shape[0] == 2:
        hw = out_ref.shape[2]
        out_ref[0] = t[:, :hw]
        out_ref[1] = t[:, hw:]
    else:
        out_ref[...] = t


def _layer(a, tp, dinv8, w_mat, b, wi, wo, split=False):
    if split:
        out_spec = pl.BlockSpec((2, RB, wo // 2), lambda i: (0, i, 0))
        out_shape = jax.ShapeDtypeStruct((2, N, wo // 2), jnp.float32)
    else:
        out_spec = _row_spec(wo)
        out_shape = jax.ShapeDtypeStruct((N, wo), jnp.float32)
    return pl.pallas_call(
        _layer_body,
        grid=(GRID,),
        in_specs=[_pair_spec(wi), _row_spec(wi), _row_spec(8),
                  _fix_spec((wi, wo)), _fix_spec((wo,))],
        out_specs=out_spec,
        out_shape=out_shape,
    )(a, tp, dinv8, w_mat, b)


def _s3_body(a_ref, t3_ref, dinv8_ref, w3_ref, b3_ref, w4_ref, t4_ref):
    di = dinv8_ref[:, 0:1]
    a = jnp.concatenate([a_ref[0], a_ref[1]], axis=1)
    t3 = jnp.concatenate([t3_ref[0], t3_ref[1]], axis=1)
    u = (a + t3) * di
    h = jnp.dot(u, w3_ref[...], preferred_element_type=jnp.float32) + b3_ref[...]
    h = jnp.maximum(h, 0.0)
    z = jnp.dot(h, w4_ref[...], preferred_element_type=jnp.float32) * di
    t4_ref[...] = jnp.concatenate([z, jnp.zeros((RB, 6), jnp.float32)], axis=1)


def _stage3(a, t3, dinv8, w3, b3, w4):
    return pl.pallas_call(
        _s3_body,
        grid=(GRID,),
        in_specs=[_pair_spec(32), _pair_spec(32), _row_spec(8),
                  _fix_spec((64, 128)), _fix_spec((128,)), _fix_spec((128, 2))],
        out_specs=_row_spec(8),
        out_shape=jax.ShapeDtypeStruct((N, 8), jnp.float32),
    )(a, t3, dinv8, w3, b3, w4)


def _s4_body(a_ref, t4_ref, dinv8_ref, b4_ref, out_ref):
    di = dinv8_ref[:, 0:1]
    v = (a_ref[0] + a_ref[1] + t4_ref[...])[:, 0:2] * di + b4_ref[...]
    m = jnp.max(v, axis=1, keepdims=True)
    e = jnp.exp(v - m)
    out_ref[...] = (v - m) - jnp.log(jnp.sum(e, axis=1, keepdims=True))


def _stage4(a, t4, dinv8, b4):
    return pl.pallas_call(
        _s4_body,
        grid=(GRID,),
        in_specs=[_pair_spec(8), _row_spec(8), _row_spec(8), _fix_spec((2,))],
        out_specs=_row_spec(2),
        out_shape=jax.ShapeDtypeStruct((N, 2), jnp.float32),
    )(a, t4, dinv8, b4)


def kernel(x, edge_index, W1, b1, W2, b2, W3, b3, W4, b4):
    src = edge_index[0].astype(jnp.int32)
    dst = edge_index[1].astype(jnp.int32)

    sp = jnp.pad(src.reshape(NCH, B), ((0, NCH_PAD - NCH), (0, 0)))
    sidx_all = jnp.stack([sp, sp + N])
    didx_all = jnp.pad(dst.reshape(NCH, B),
                       ((0, NCH_PAD - NCH), (0, 0)))[None]
    sidx_all, didx_all = lax.optimization_barrier((sidx_all, didx_all))

    dp, sidx_all, didx_all = _agg(jnp.ones((B, 8), jnp.float32),
                                  sidx_all, didx_all, 8, hist=True)
    t1, dinv8 = _stage0(dp, x)

    a1 = _agg(t1, sidx_all, didx_all, 8)
    t2 = _layer(a1, t1, dinv8, jnp.pad(W1, ((0, 2), (0, 0))), b1, 8, 32)

    a2 = _agg(t2, sidx_all, didx_all, 32)
    t3s = _layer(a2, t2, dinv8, W2, b2, 32, 64, split=True)

    a3 = _agg(t3s.reshape(2 * N, 32), sidx_all, didx_all, 32, full=True)
    t4 = _stage3(a3, t3s, dinv8, W3, b3, W4)

    a4 = _agg(t4, sidx_all, didx_all, 8)
    return _stage4(a4, t4, dinv8, b4)
